# baseline (speedup 1.0000x reference)
import functools

import jax
import jax.numpy as jnp
from jax import lax
from jax.experimental import pallas as pl
from jax.experimental.pallas import tpu as pltpu
from jax.experimental.pallas import tpu_sc as plsc

F32 = jnp.float32


def _emb_body(x_ref, w_ref, b_ref, out_ref):
    z = jnp.dot(x_ref[...], w_ref[...], preferred_element_type=F32) + b_ref[...]
    out_ref[0] = z[:, :128]
    out_ref[1] = z[:, 128:]


def _conv_body(a_ref, h_ref, wl_ref, ws_ref, b_ref, out_ref):
    z = (jnp.dot(a_ref[0], wl_ref[0], preferred_element_type=F32)
         + jnp.dot(a_ref[1], wl_ref[1], preferred_element_type=F32)
         + jnp.dot(h_ref[0], ws_ref[0], preferred_element_type=F32)
         + jnp.dot(h_ref[1], ws_ref[1], preferred_element_type=F32)
         + b_ref[...])
    nrm = jnp.sqrt(jnp.sum(z * z, axis=1, keepdims=True))
    zn = z / jnp.maximum(nrm, 1e-12)
    act = jnp.maximum(zn, 0.1 * zn)
    out_ref[0] = act[:, :128]
    out_ref[1] = act[:, 128:]


def _pool_body(h_ref, b3_ref, wp_ref, bp_ref, sums_ref, cnts_ref, *, bsz, ngraphs):
    i = pl.program_id(0)
    p = (jnp.dot(h_ref[0], wp_ref[0], preferred_element_type=F32)
         + jnp.dot(h_ref[1], wp_ref[1], preferred_element_type=F32)
         + bp_ref[...])
    p = jnp.maximum(p, 0.1 * p)
    b = b3_ref[...].reshape(1, bsz)
    ohT = (lax.broadcasted_iota(jnp.int32, (ngraphs, bsz), 0)
           == jnp.broadcast_to(b, (ngraphs, bsz))).astype(F32)
    s = jnp.dot(ohT, p, preferred_element_type=F32)
    c = jnp.broadcast_to(jnp.sum(ohT, axis=1, keepdims=True), (ngraphs, 128))

    @pl.when(i == 0)
    def _():
        sums_ref[...] = s
        cnts_ref[...] = c

    @pl.when(i > 0)
    def _():
        sums_ref[...] += s
        cnts_ref[...] += c


def _head_body(s_ref, c_ref, w0_ref, b0_ref, w1_ref, b1_ref, w2_ref, b2_ref,
               out_ref, *, ngraphs, nclass):
    g = s_ref[...] / jnp.maximum(c_ref[...], 1.0)
    a = jnp.dot(g, w0_ref[...], preferred_element_type=F32) + b0_ref[...]
    a = jnp.maximum(a, 0.0)
    a = jnp.dot(a, w1_ref[...], preferred_element_type=F32) + b1_ref[...]
    a = jnp.maximum(a, 0.0)
    z = jnp.dot(a, w2_ref[...], preferred_element_type=F32) + b2_ref[...]
    mask = lax.broadcasted_iota(jnp.int32, (ngraphs, 128), 1) < nclass
    z = jnp.where(mask, z, -1e30)
    m = jnp.max(z, axis=1, keepdims=True)
    e = jnp.exp(z - m)
    out_ref[...] = e / jnp.sum(e, axis=1, keepdims=True)


def _make_segsum(N, E):
    info = plsc.get_sparse_core_info()
    NS = info.num_subcores
    CH = 64
    e_raw = E // NS
    n_ch = -(-e_raw // CH)
    if n_ch % 2 == 0:
        n_ch += 1
    e_pt = n_ch * CH
    NA = N + 16
    rpt = (NA // NS) // 8 * 8
    remz = NA - NS * rpt
    remo = N - NS * rpt
    mesh = plsc.VectorSubcoreMesh(core_axis_name="c", subcore_axis_name="s")
    assert E % NS == 0 and NA % 16 == 0 and remz >= 0 and remo >= 0

    @functools.partial(
        pl.kernel, mesh=mesh,
        out_type=jax.ShapeDtypeStruct((2 * N, 128), F32),
        scratch_types=[
            pltpu.VMEM((e_pt,), jnp.int32),
            pltpu.VMEM((n_ch, CH), jnp.int32),
            pltpu.VMEM((CH, 128), F32),
            pltpu.VMEM((CH, 128), F32),
            pltpu.VMEM_SHARED((NA, 128), F32),
            pltpu.SemaphoreType.DMA,
            pltpu.SemaphoreType.DMA,
            pltpu.SemaphoreType.DMA,
        ],
    )
    def seg(h_hbm, src_hbm, dst3_hbm, zero_hbm, out_hbm,
            srcv, dstv, rows_a, rows_b, acc, sem_a, sem_b, sem_z):
        c = lax.axis_index("c")
        s = lax.axis_index("s")
        cN = c * N
        pltpu.async_copy(zero_hbm.at[pl.ds(0, rpt)],
                         acc.at[pl.ds(s * rpt, rpt)], sem_z)

        @pl.when(s == NS - 1)
        def _():
            pltpu.async_copy(zero_hbm.at[pl.ds(0, remz)],
                             acc.at[pl.ds(NS * rpt, remz)], sem_z)

        pltpu.sync_copy(src_hbm.at[pl.ds(s * e_pt, e_pt)], srcv)
        pltpu.sync_copy(dst3_hbm.at[s], dstv)

        def adj(j, carry):
            srcv[pl.ds(j * 16, 16)] = srcv[pl.ds(j * 16, 16)] + cN
            return carry

        lax.fori_loop(0, e_pt // 16, adj, 0)
        pltpu.make_async_copy(zero_hbm.at[pl.ds(0, rpt)],
                              acc.at[pl.ds(s * rpt, rpt)], sem_z).wait()

        @pl.when(s == NS - 1)
        def _():
            pltpu.make_async_copy(zero_hbm.at[pl.ds(0, remz)],
                                  acc.at[pl.ds(NS * rpt, remz)], sem_z).wait()

        plsc.subcore_barrier()

        def gather(i, buf, sem):
            pltpu.async_copy(h_hbm.at[srcv.at[pl.ds(i * CH, CH)]], buf, sem)

        def gwait(buf, sem):
            pltpu.make_async_copy(h_hbm.at[pl.ds(0, CH)], buf, sem).wait()

        def scat(i, buf):
            pltpu.sync_copy(buf, acc.at[dstv.at[i]], add=True)

        gather(0, rows_a, sem_a)

        def body(k, carry):
            i0 = k * 2
            gather(i0 + 1, rows_b, sem_b)
            gwait(rows_a, sem_a)
            scat(i0, rows_a)
            gather(i0 + 2, rows_a, sem_a)
            gwait(rows_b, sem_b)
            scat(i0 + 1, rows_b)
            return carry

        lax.fori_loop(0, (n_ch - 1) // 2, body, 0)
        gwait(rows_a, sem_a)
        scat(n_ch - 1, rows_a)
        plsc.subcore_barrier()
        pltpu.sync_copy(acc.at[pl.ds(s * rpt, rpt)],
                        out_hbm.at[pl.ds(cN + s * rpt, rpt)])

        @pl.when(s == NS - 1)
        def _():
            pltpu.sync_copy(acc.at[pl.ds(NS * rpt, remo)],
                            out_hbm.at[pl.ds(cN + NS * rpt, remo)])

    def prep(src, dst):
        pad = e_pt - e_raw
        src_p = jnp.pad(src.reshape(NS, e_raw), ((0, 0), (0, pad))).reshape(-1)
        dummy = (N + jnp.arange(NS, dtype=jnp.int32))[:, None]
        dst_p = jnp.concatenate(
            [dst.reshape(NS, e_raw),
             jnp.broadcast_to(dummy, (NS, pad))], axis=1).reshape(NS, n_ch, CH)
        return src_p, dst_p

    return seg, prep


def kernel(x, edge_index, batch, W_emb, b_emb, Wl0, Wself0, bconv0,
           Wl1, Wself1, bconv1, Wl2, Wself2, bconv2, W_p, b_p,
           W_a0, b_a0, W_a1, b_a1, W_a2, b_a2):
    N, D = x.shape
    E = edge_index.shape[1]
    G = 32
    NC = W_a2.shape[0]
    B = 1000
    grid = N // B

    src = edge_index[0]
    dst = edge_index[1]
    zeros_blk = jnp.zeros(((N // 16) // 8 * 8, 128), F32)

    h_split = pl.pallas_call(
        _emb_body,
        grid=(grid,),
        in_specs=[
            pl.BlockSpec((B, D), lambda i: (i, 0)),
            pl.BlockSpec((D, D), lambda i: (0, 0)),
            pl.BlockSpec((1, D), lambda i: (0, 0)),
        ],
        out_specs=pl.BlockSpec((2, B, 128), lambda i: (0, i, 0)),
        out_shape=jax.ShapeDtypeStruct((2, N, 128), F32),
    )(x, W_emb.T, b_emb.reshape(1, D))

    segsum, seg_prep = _make_segsum(N, E)
    src_p, dst_p = seg_prep(src, dst)

    conv_call = pl.pallas_call(
        _conv_body,
        grid=(grid,),
        in_specs=[
            pl.BlockSpec((2, B, 128), lambda i: (0, i, 0)),
            pl.BlockSpec((2, B, 128), lambda i: (0, i, 0)),
            pl.BlockSpec((2, 128, D), lambda i: (0, 0, 0)),
            pl.BlockSpec((2, 128, D), lambda i: (0, 0, 0)),
            pl.BlockSpec((1, D), lambda i: (0, 0)),
        ],
        out_specs=pl.BlockSpec((2, B, 128), lambda i: (0, i, 0)),
        out_shape=jax.ShapeDtypeStruct((2, N, 128), F32),
    )

    for Wl, Ws, bc in ((Wl0, Wself0, bconv0), (Wl1, Wself1, bconv1),
                       (Wl2, Wself2, bconv2)):
        agg2 = segsum(h_split.reshape(2 * N, 128), src_p, dst_p, zeros_blk)
        h_split = conv_call(
            agg2.reshape(2, N, 128), h_split,
            Wl.T.reshape(2, 128, D), Ws.T.reshape(2, 128, D),
            bc.reshape(1, D))

    sums, cnts = pl.pallas_call(
        functools.partial(_pool_body, bsz=B, ngraphs=G),
        grid=(grid,),
        in_specs=[
            pl.BlockSpec((2, B, 128), lambda i: (0, i, 0)),
            pl.BlockSpec((1, 1, B), lambda i: (i, 0, 0)),
            pl.BlockSpec((2, 128, 128), lambda i: (0, 0, 0)),
            pl.BlockSpec((1, 128), lambda i: (0, 0)),
        ],
        out_specs=[
            pl.BlockSpec((G, 128), lambda i: (0, 0)),
            pl.BlockSpec((G, 128), lambda i: (0, 0)),
        ],
        out_shape=[
            jax.ShapeDtypeStruct((G, 128), F32),
            jax.ShapeDtypeStruct((G, 128), F32),
        ],
    )(h_split, batch.reshape(grid, 1, B), W_p.T.reshape(2, 128, 128),
      b_p.reshape(1, 128))

    w1T = jnp.zeros((128, 128), F32).at[:, :W_a1.shape[0]].set(W_a1.T)
    b1 = jnp.zeros((1, 128), F32).at[0, :W_a1.shape[0]].set(b_a1)
    w2T = jnp.zeros((128, 128), F32).at[:W_a2.shape[1], :NC].set(W_a2.T)
    b2 = jnp.zeros((1, 128), F32).at[0, :NC].set(b_a2)

    out = pl.pallas_call(
        functools.partial(_head_body, ngraphs=G, nclass=NC),
        grid=(1,),
        in_specs=[pl.BlockSpec((G, 128), lambda i: (0, 0))] * 2
                 + [pl.BlockSpec((128, 128), lambda i: (0, 0)),
                    pl.BlockSpec((1, 128), lambda i: (0, 0))] * 3,
        out_specs=pl.BlockSpec((G, 128), lambda i: (0, 0)),
        out_shape=jax.ShapeDtypeStruct((G, 128), F32),
    )(sums, cnts, W_a0.T, b_a0.reshape(1, 128), w1T, b1, w2T, b2)

    return out[:, :NC]


# speedup vs baseline: 1.2292x; 1.2292x over previous
import functools

import jax
import jax.numpy as jnp
from jax import lax
from jax.experimental import pallas as pl
from jax.experimental.pallas import tpu as pltpu
from jax.experimental.pallas import tpu_sc as plsc

F32 = jnp.float32


def _dot_t(a, w):
    return lax.dot_general(a, w, (((1,), (1,)), ((), ())),
                           preferred_element_type=F32)


def _emb_body(x_ref, w_ref, b_ref, out_ref):
    z = _dot_t(x_ref[...], w_ref[...]) + b_ref[...]
    out_ref[0] = z[:, :128]
    out_ref[1] = z[:, 128:]


def _conv_body(a_ref, h_ref, wl_ref, ws_ref, b_ref, out_ref):
    z = (_dot_t(a_ref[0], wl_ref[:, :128])
         + _dot_t(a_ref[1], wl_ref[:, 128:])
         + _dot_t(h_ref[0], ws_ref[:, :128])
         + _dot_t(h_ref[1], ws_ref[:, 128:])
         + b_ref[...])
    nrm = jnp.sqrt(jnp.sum(z * z, axis=1, keepdims=True))
    zn = z / jnp.maximum(nrm, 1e-12)
    act = jnp.maximum(zn, 0.1 * zn)
    out_ref[0] = act[:, :128]
    out_ref[1] = act[:, 128:]


def _pool_body(h_ref, b3_ref, wp_ref, bp_ref, sums_ref, cnts_ref, *, bsz, ngraphs):
    i = pl.program_id(0)
    p = (_dot_t(h_ref[0], wp_ref[:, :128])
         + _dot_t(h_ref[1], wp_ref[:, 128:])
         + bp_ref[...])
    p = jnp.maximum(p, 0.1 * p)
    b = b3_ref[...].reshape(1, bsz)
    ohT = (lax.broadcasted_iota(jnp.int32, (ngraphs, bsz), 0)
           == jnp.broadcast_to(b, (ngraphs, bsz))).astype(F32)
    s = jnp.dot(ohT, p, preferred_element_type=F32)
    c = jnp.broadcast_to(jnp.sum(ohT, axis=1, keepdims=True), (ngraphs, 128))

    @pl.when(i == 0)
    def _():
        sums_ref[...] = s
        cnts_ref[...] = c

    @pl.when(i > 0)
    def _():
        sums_ref[...] += s
        cnts_ref[...] += c


def _head_body(s_ref, c_ref, w0_ref, b0_ref, w1_ref, b1_ref, w2_ref, b2_ref,
               out_ref, *, ngraphs, nclass):
    g = s_ref[...] / jnp.maximum(c_ref[...], 1.0)
    a = jnp.maximum(_dot_t(g, w0_ref[...]) + b0_ref[...], 0.0)
    a = jnp.maximum(_dot_t(a, w1_ref[...]) + b1_ref[...], 0.0)
    z = _dot_t(a, w2_ref[...]) + b2_ref[...]
    m = jnp.max(z, axis=1, keepdims=True)
    e = jnp.exp(z - m)
    out_ref[...] = e / jnp.sum(e, axis=1, keepdims=True)


def _make_segsum(N, E):
    info = plsc.get_sparse_core_info()
    NS = info.num_subcores
    CH = 80
    e_raw = E // NS
    n_ch = -(-e_raw // CH)
    if n_ch % 2 == 0:
        n_ch += 1
    e_pt = n_ch * CH
    NA = N + 16
    rpt = (NA // NS) // 8 * 8
    remz = NA - NS * rpt
    remo = N - NS * rpt
    mesh = plsc.VectorSubcoreMesh(core_axis_name="c", subcore_axis_name="s")
    assert E % NS == 0 and NA % 16 == 0 and remz >= 0 and remo >= 0

    @functools.partial(
        pl.kernel, mesh=mesh,
        out_type=jax.ShapeDtypeStruct((2 * N, 128), F32),
        scratch_types=[
            pltpu.VMEM((e_pt,), jnp.int32),
            pltpu.VMEM((n_ch, CH), jnp.int32),
            pltpu.VMEM((CH, 128), F32),
            pltpu.VMEM((CH, 128), F32),
            pltpu.VMEM_SHARED((NA, 128), F32),
            pltpu.SemaphoreType.DMA,
            pltpu.SemaphoreType.DMA,
            pltpu.SemaphoreType.DMA,
        ],
    )
    def seg(h_hbm, src_hbm, dst3_hbm, zero_hbm, out_hbm,
            srcv, dstv, rows_a, rows_b, acc, sem_a, sem_b, sem_z):
        c = lax.axis_index("c")
        s = lax.axis_index("s")
        cN = c * N
        pltpu.async_copy(zero_hbm.at[pl.ds(0, rpt)],
                         acc.at[pl.ds(s * rpt, rpt)], sem_z)

        @pl.when(s == NS - 1)
        def _():
            pltpu.async_copy(zero_hbm.at[pl.ds(0, remz)],
                             acc.at[pl.ds(NS * rpt, remz)], sem_z)

        pltpu.sync_copy(src_hbm.at[pl.ds(s * e_pt, e_pt)], srcv)
        pltpu.sync_copy(dst3_hbm.at[s], dstv)

        def adj(j, carry):
            srcv[pl.ds(j * 16, 16)] = srcv[pl.ds(j * 16, 16)] + cN
            return carry

        lax.fori_loop(0, e_pt // 16, adj, 0)
        pltpu.make_async_copy(zero_hbm.at[pl.ds(0, rpt)],
                              acc.at[pl.ds(s * rpt, rpt)], sem_z).wait()

        @pl.when(s == NS - 1)
        def _():
            pltpu.make_async_copy(zero_hbm.at[pl.ds(0, remz)],
                                  acc.at[pl.ds(NS * rpt, remz)], sem_z).wait()

        plsc.subcore_barrier()

        def gather(i, buf, sem):
            pltpu.async_copy(h_hbm.at[srcv.at[pl.ds(i * CH, CH)]], buf, sem)

        def gwait(buf, sem):
            pltpu.make_async_copy(h_hbm.at[pl.ds(0, CH)], buf, sem).wait()

        def scat(i, buf):
            pltpu.sync_copy(buf, acc.at[dstv.at[i]], add=True)

        gather(0, rows_a, sem_a)

        def body(k, carry):
            i0 = k * 2
            gather(i0 + 1, rows_b, sem_b)
            gwait(rows_a, sem_a)
            scat(i0, rows_a)
            gather(i0 + 2, rows_a, sem_a)
            gwait(rows_b, sem_b)
            scat(i0 + 1, rows_b)
            return carry

        lax.fori_loop(0, (n_ch - 1) // 2, body, 0)
        gwait(rows_a, sem_a)
        scat(n_ch - 1, rows_a)
        plsc.subcore_barrier()
        pltpu.sync_copy(acc.at[pl.ds(s * rpt, rpt)],
                        out_hbm.at[pl.ds(cN + s * rpt, rpt)])

        @pl.when(s == NS - 1)
        def _():
            pltpu.sync_copy(acc.at[pl.ds(NS * rpt, remo)],
                            out_hbm.at[pl.ds(cN + NS * rpt, remo)])

    def prep(src, dst):
        pad = e_pt - e_raw
        src_p = jnp.pad(src.reshape(NS, e_raw), ((0, 0), (0, pad))).reshape(-1)
        dummy = (N + jnp.arange(NS, dtype=jnp.int32))[:, None]
        dst_p = jnp.concatenate(
            [dst.reshape(NS, e_raw),
             jnp.broadcast_to(dummy, (NS, pad))], axis=1).reshape(NS, n_ch, CH)
        return src_p, dst_p

    return seg, prep


def kernel(x, edge_index, batch, W_emb, b_emb, Wl0, Wself0, bconv0,
           Wl1, Wself1, bconv1, Wl2, Wself2, bconv2, W_p, b_p,
           W_a0, b_a0, W_a1, b_a1, W_a2, b_a2):
    N, D = x.shape
    E = edge_index.shape[1]
    G = 32
    NC = W_a2.shape[0]
    B = 1000
    grid = N // B

    src = edge_index[0]
    dst = edge_index[1]
    zeros_blk = jnp.zeros(((N // 16) // 8 * 8, 128), F32)

    h_split = pl.pallas_call(
        _emb_body,
        grid=(grid,),
        in_specs=[
            pl.BlockSpec((B, D), lambda i: (i, 0)),
            pl.BlockSpec((D, D), lambda i: (0, 0)),
            pl.BlockSpec((1, D), lambda i: (0, 0)),
        ],
        out_specs=pl.BlockSpec((2, B, 128), lambda i: (0, i, 0)),
        out_shape=jax.ShapeDtypeStruct((2, N, 128), F32),
    )(x, W_emb, b_emb.reshape(1, D))

    segsum, seg_prep = _make_segsum(N, E)
    src_p, dst_p = seg_prep(src, dst)

    conv_call = pl.pallas_call(
        _conv_body,
        grid=(grid,),
        in_specs=[
            pl.BlockSpec((2, B, 128), lambda i: (0, i, 0)),
            pl.BlockSpec((2, B, 128), lambda i: (0, i, 0)),
            pl.BlockSpec((D, D), lambda i: (0, 0)),
            pl.BlockSpec((D, D), lambda i: (0, 0)),
            pl.BlockSpec((1, D), lambda i: (0, 0)),
        ],
        out_specs=pl.BlockSpec((2, B, 128), lambda i: (0, i, 0)),
        out_shape=jax.ShapeDtypeStruct((2, N, 128), F32),
    )

    for Wl, Ws, bc in ((Wl0, Wself0, bconv0), (Wl1, Wself1, bconv1),
                       (Wl2, Wself2, bconv2)):
        agg2 = segsum(h_split.reshape(2 * N, 128), src_p, dst_p, zeros_blk)
        h_split = conv_call(
            agg2.reshape(2, N, 128), h_split, Wl, Ws, bc.reshape(1, D))

    sums, cnts = pl.pallas_call(
        functools.partial(_pool_body, bsz=B, ngraphs=G),
        grid=(grid,),
        in_specs=[
            pl.BlockSpec((2, B, 128), lambda i: (0, i, 0)),
            pl.BlockSpec((1, 1, B), lambda i: (i, 0, 0)),
            pl.BlockSpec((128, D), lambda i: (0, 0)),
            pl.BlockSpec((1, 128), lambda i: (0, 0)),
        ],
        out_specs=[
            pl.BlockSpec((G, 128), lambda i: (0, 0)),
            pl.BlockSpec((G, 128), lambda i: (0, 0)),
        ],
        out_shape=[
            jax.ShapeDtypeStruct((G, 128), F32),
            jax.ShapeDtypeStruct((G, 128), F32),
        ],
    )(h_split, batch.reshape(grid, 1, B), W_p, b_p.reshape(1, 128))

    A0, A1 = W_a0.shape[0], W_a1.shape[0]
    out = pl.pallas_call(
        functools.partial(_head_body, ngraphs=G, nclass=NC),
        grid=(1,),
        in_specs=[pl.BlockSpec((G, 128), lambda i: (0, 0))] * 2
                 + [pl.BlockSpec((A0, 128), lambda i: (0, 0)),
                    pl.BlockSpec((1, A0), lambda i: (0, 0)),
                    pl.BlockSpec((A1, A0), lambda i: (0, 0)),
                    pl.BlockSpec((1, A1), lambda i: (0, 0)),
                    pl.BlockSpec((NC, A1), lambda i: (0, 0)),
                    pl.BlockSpec((1, NC), lambda i: (0, 0))],
        out_specs=pl.BlockSpec((G, NC), lambda i: (0, 0)),
        out_shape=jax.ShapeDtypeStruct((G, NC), F32),
    )(sums, cnts, W_a0, b_a0.reshape(1, A0), W_a1, b_a1.reshape(1, A1),
      W_a2, b_a2.reshape(1, NC))

    return out


# speedup vs baseline: 1.4128x; 1.1494x over previous
import functools

import jax
import jax.numpy as jnp
from jax import lax
from jax.experimental import pallas as pl
from jax.experimental.pallas import tpu as pltpu
from jax.experimental.pallas import tpu_sc as plsc

F32 = jnp.float32


def _dot_t(a, w):
    return lax.dot_general(a, w, (((1,), (1,)), ((), ())),
                           preferred_element_type=F32)


def _emb_body(x_ref, w_ref, b_ref, out_ref):
    z = _dot_t(x_ref[...], w_ref[...]) + b_ref[...]
    out_ref[0] = z[:, :128]
    out_ref[1] = z[:, 128:]


def _conv_body(a_ref, h_ref, wl_ref, ws_ref, b_ref, out_ref):
    z = (_dot_t(a_ref[0], wl_ref[:, :128])
         + _dot_t(a_ref[1], wl_ref[:, 128:])
         + _dot_t(h_ref[0], ws_ref[:, :128])
         + _dot_t(h_ref[1], ws_ref[:, 128:])
         + b_ref[...])
    nrm = jnp.sqrt(jnp.sum(z * z, axis=1, keepdims=True))
    zn = z / jnp.maximum(nrm, 1e-12)
    act = jnp.maximum(zn, 0.1 * zn)
    out_ref[0] = act[:, :128]
    out_ref[1] = act[:, 128:]


def _pool_body(h_ref, b3_ref, wp_ref, bp_ref, sums_ref, cnts_ref, *, bsz, ngraphs):
    i = pl.program_id(0)
    p = (_dot_t(h_ref[0], wp_ref[:, :128])
         + _dot_t(h_ref[1], wp_ref[:, 128:])
         + bp_ref[...])
    p = jnp.maximum(p, 0.1 * p)
    b = b3_ref[...].reshape(1, bsz)
    ohT = (lax.broadcasted_iota(jnp.int32, (ngraphs, bsz), 0)
           == jnp.broadcast_to(b, (ngraphs, bsz))).astype(F32)
    s = jnp.dot(ohT, p, preferred_element_type=F32)
    c = jnp.broadcast_to(jnp.sum(ohT, axis=1, keepdims=True), (ngraphs, 128))

    @pl.when(i == 0)
    def _():
        sums_ref[...] = s
        cnts_ref[...] = c

    @pl.when(i > 0)
    def _():
        sums_ref[...] += s
        cnts_ref[...] += c


def _head_body(s_ref, c_ref, w0_ref, b0_ref, w1_ref, b1_ref, w2_ref, b2_ref,
               out_ref, *, ngraphs, nclass):
    g = s_ref[...] / jnp.maximum(c_ref[...], 1.0)
    a = jnp.maximum(_dot_t(g, w0_ref[...]) + b0_ref[...], 0.0)
    a = jnp.maximum(_dot_t(a, w1_ref[...]) + b1_ref[...], 0.0)
    z = _dot_t(a, w2_ref[...]) + b2_ref[...]
    m = jnp.max(z, axis=1, keepdims=True)
    e = jnp.exp(z - m)
    out_ref[...] = e / jnp.sum(e, axis=1, keepdims=True)


def _make_segsum(N, E):
    info = plsc.get_sparse_core_info()
    NS = info.num_subcores
    CH = 80
    e_raw = E // NS
    n_ch = -(-e_raw // CH)
    while n_ch % 3 != 2:
        n_ch += 1
    e_pt = n_ch * CH
    NA = N + 16
    rpt = (NA // NS) // 8 * 8
    remz = NA - NS * rpt
    remo = N - NS * rpt
    mesh = plsc.VectorSubcoreMesh(core_axis_name="c", subcore_axis_name="s")
    assert E % NS == 0 and NA % 16 == 0 and remz >= 0 and remo >= 0

    @functools.partial(
        pl.kernel, mesh=mesh,
        out_type=jax.ShapeDtypeStruct((2 * N, 128), F32),
        scratch_types=[
            pltpu.VMEM((e_pt,), jnp.int32),
            pltpu.VMEM((CH,), jnp.int32),
            pltpu.VMEM((CH,), jnp.int32),
            pltpu.VMEM((CH,), jnp.int32),
            pltpu.VMEM((CH, 128), F32),
            pltpu.VMEM((CH, 128), F32),
            pltpu.VMEM((CH, 128), F32),
            pltpu.SemaphoreType.DMA,
            pltpu.SemaphoreType.DMA,
            pltpu.SemaphoreType.DMA,
            pltpu.SemaphoreType.DMA,
            pltpu.SemaphoreType.DMA,
            pltpu.SemaphoreType.DMA,
            pltpu.SemaphoreType.DMA,
            pltpu.SemaphoreType.DMA,
            pltpu.SemaphoreType.DMA,
            pltpu.VMEM_SHARED((NA, 128), F32),
            pltpu.SemaphoreType.DMA,
        ],
    )
    def seg(h_hbm, src_hbm, dst_hbm, zero_hbm, out_hbm,
            srcv, d0, d1, d2, b0, b1, b2,
            sg0, sg1, sg2, ss0, ss1, ss2, sd0, sd1, sd2, acc, sem_z):
        c = lax.axis_index("c")
        s = lax.axis_index("s")
        cN = c * N
        pltpu.async_copy(zero_hbm.at[pl.ds(0, rpt)],
                         acc.at[pl.ds(s * rpt, rpt)], sem_z)

        @pl.when(s == NS - 1)
        def _():
            pltpu.async_copy(zero_hbm.at[pl.ds(0, remz)],
                             acc.at[pl.ds(NS * rpt, remz)], sem_z)

        pltpu.sync_copy(src_hbm.at[pl.ds((c * NS + s) * e_pt, e_pt)], srcv)
        pltpu.make_async_copy(zero_hbm.at[pl.ds(0, rpt)],
                              acc.at[pl.ds(s * rpt, rpt)], sem_z).wait()

        @pl.when(s == NS - 1)
        def _():
            pltpu.make_async_copy(zero_hbm.at[pl.ds(0, remz)],
                                  acc.at[pl.ds(NS * rpt, remz)], sem_z).wait()

        plsc.subcore_barrier()
        dbase = s * e_pt

        def gather(i, buf, sem, dstg, dsem):
            pltpu.async_copy(h_hbm.at[srcv.at[pl.ds(i * CH, CH)]], buf, sem)
            pltpu.async_copy(dst_hbm.at[pl.ds(dbase + i * CH, CH)], dstg, dsem)

        def gwait(buf, sem, dstg, dsem):
            pltpu.make_async_copy(h_hbm.at[pl.ds(0, CH)], buf, sem).wait()
            pltpu.make_async_copy(dst_hbm.at[pl.ds(0, CH)], dstg, dsem).wait()

        def sstart(i, buf, dstg, sem):
            pltpu.async_copy(buf, acc.at[dstg], sem, add=True)

        def swait(buf, sem):
            pltpu.make_async_copy(buf, acc.at[d0], sem).wait()

        K = (n_ch - 2) // 3
        gather(0, b0, sg0, d0, sd0)
        gather(1, b1, sg1, d1, sd1)
        gwait(b0, sg0, d0, sd0)
        sstart(0, b0, d0, ss0)
        gather(2, b2, sg2, d2, sd2)
        gwait(b1, sg1, d1, sd1)
        sstart(1, b1, d1, ss1)

        def body(k, carry):
            j = 3 * k + 2
            swait(b0, ss0)
            gather(j + 1, b0, sg0, d0, sd0)
            gwait(b2, sg2, d2, sd2)
            sstart(j, b2, d2, ss2)
            swait(b1, ss1)
            gather(j + 2, b1, sg1, d1, sd1)
            gwait(b0, sg0, d0, sd0)
            sstart(j + 1, b0, d0, ss0)
            swait(b2, ss2)

            @pl.when(k < K - 1)
            def _():
                gather(j + 3, b2, sg2, d2, sd2)

            gwait(b1, sg1, d1, sd1)
            sstart(j + 2, b1, d1, ss1)
            return carry

        lax.fori_loop(0, K, body, 0)
        swait(b0, ss0)
        swait(b1, ss1)
        plsc.subcore_barrier()
        pltpu.sync_copy(acc.at[pl.ds(s * rpt, rpt)],
                        out_hbm.at[pl.ds(cN + s * rpt, rpt)])

        @pl.when(s == NS - 1)
        def _():
            pltpu.sync_copy(acc.at[pl.ds(NS * rpt, remo)],
                            out_hbm.at[pl.ds(cN + NS * rpt, remo)])

    def prep(src, dst):
        pad = e_pt - e_raw
        spread = (jnp.arange(NS, dtype=jnp.int32)[:, None] * max(pad, 1)
                  + jnp.arange(pad, dtype=jnp.int32)[None, :]) % N
        src_p = jnp.concatenate([src.reshape(NS, e_raw), spread],
                                axis=1).reshape(-1)
        src_p = jnp.concatenate([src_p, src_p + N])
        dummy = (N + jnp.arange(NS, dtype=jnp.int32))[:, None]
        dst_p = jnp.concatenate(
            [dst.reshape(NS, e_raw),
             jnp.broadcast_to(dummy, (NS, pad))], axis=1).reshape(-1)
        return src_p, dst_p

    return seg, prep


def kernel(x, edge_index, batch, W_emb, b_emb, Wl0, Wself0, bconv0,
           Wl1, Wself1, bconv1, Wl2, Wself2, bconv2, W_p, b_p,
           W_a0, b_a0, W_a1, b_a1, W_a2, b_a2):
    N, D = x.shape
    E = edge_index.shape[1]
    G = 32
    NC = W_a2.shape[0]
    B = 1000
    grid = N // B

    src = edge_index[0]
    dst = edge_index[1]
    zeros_blk = jnp.zeros(((N // 16) // 8 * 8, 128), F32)

    h_split = pl.pallas_call(
        _emb_body,
        grid=(grid,),
        in_specs=[
            pl.BlockSpec((B, D), lambda i: (i, 0)),
            pl.BlockSpec((D, D), lambda i: (0, 0)),
            pl.BlockSpec((1, D), lambda i: (0, 0)),
        ],
        out_specs=pl.BlockSpec((2, B, 128), lambda i: (0, i, 0)),
        out_shape=jax.ShapeDtypeStruct((2, N, 128), F32),
    )(x, W_emb, b_emb.reshape(1, D))

    segsum, seg_prep = _make_segsum(N, E)
    src_p, dst_p = seg_prep(src, dst)

    conv_call = pl.pallas_call(
        _conv_body,
        grid=(grid,),
        in_specs=[
            pl.BlockSpec((2, B, 128), lambda i: (0, i, 0)),
            pl.BlockSpec((2, B, 128), lambda i: (0, i, 0)),
            pl.BlockSpec((D, D), lambda i: (0, 0)),
            pl.BlockSpec((D, D), lambda i: (0, 0)),
            pl.BlockSpec((1, D), lambda i: (0, 0)),
        ],
        out_specs=pl.BlockSpec((2, B, 128), lambda i: (0, i, 0)),
        out_shape=jax.ShapeDtypeStruct((2, N, 128), F32),
    )

    for Wl, Ws, bc in ((Wl0, Wself0, bconv0), (Wl1, Wself1, bconv1),
                       (Wl2, Wself2, bconv2)):
        agg2 = segsum(h_split.reshape(2 * N, 128), src_p, dst_p, zeros_blk)
        h_split = conv_call(
            agg2.reshape(2, N, 128), h_split, Wl, Ws, bc.reshape(1, D))

    sums, cnts = pl.pallas_call(
        functools.partial(_pool_body, bsz=B, ngraphs=G),
        grid=(grid,),
        in_specs=[
            pl.BlockSpec((2, B, 128), lambda i: (0, i, 0)),
            pl.BlockSpec((1, 1, B), lambda i: (i, 0, 0)),
            pl.BlockSpec((128, D), lambda i: (0, 0)),
            pl.BlockSpec((1, 128), lambda i: (0, 0)),
        ],
        out_specs=[
            pl.BlockSpec((G, 128), lambda i: (0, 0)),
            pl.BlockSpec((G, 128), lambda i: (0, 0)),
        ],
        out_shape=[
            jax.ShapeDtypeStruct((G, 128), F32),
            jax.ShapeDtypeStruct((G, 128), F32),
        ],
    )(h_split, batch.reshape(grid, 1, B), W_p, b_p.reshape(1, 128))

    A0, A1 = W_a0.shape[0], W_a1.shape[0]
    out = pl.pallas_call(
        functools.partial(_head_body, ngraphs=G, nclass=NC),
        grid=(1,),
        in_specs=[pl.BlockSpec((G, 128), lambda i: (0, 0))] * 2
                 + [pl.BlockSpec((A0, 128), lambda i: (0, 0)),
                    pl.BlockSpec((1, A0), lambda i: (0, 0)),
                    pl.BlockSpec((A1, A0), lambda i: (0, 0)),
                    pl.BlockSpec((1, A1), lambda i: (0, 0)),
                    pl.BlockSpec((NC, A1), lambda i: (0, 0)),
                    pl.BlockSpec((1, NC), lambda i: (0, 0))],
        out_specs=pl.BlockSpec((G, NC), lambda i: (0, 0)),
        out_shape=jax.ShapeDtypeStruct((G, NC), F32),
    )(sums, cnts, W_a0, b_a0.reshape(1, A0), W_a1, b_a1.reshape(1, A1),
      W_a2, b_a2.reshape(1, NC))

    return out


# speedup vs baseline: 1.4570x; 1.0313x over previous
import functools

import jax
import jax.numpy as jnp
from jax import lax
from jax.experimental import pallas as pl
from jax.experimental.pallas import tpu as pltpu
from jax.experimental.pallas import tpu_sc as plsc

F32 = jnp.float32


def _dot_t(a, w):
    return lax.dot_general(a, w, (((1,), (1,)), ((), ())),
                           preferred_element_type=F32)


def _emb_body(x_ref, w_ref, b_ref, out_ref):
    z = _dot_t(x_ref[...], w_ref[...]) + b_ref[...]
    out_ref[0] = z[:, :128]
    out_ref[1] = z[:, 128:]


def _conv_body(a_ref, h_ref, wl_ref, ws_ref, b_ref, out_ref):
    z = (_dot_t(a_ref[0], wl_ref[:, :128])
         + _dot_t(a_ref[1], wl_ref[:, 128:])
         + _dot_t(h_ref[0], ws_ref[:, :128])
         + _dot_t(h_ref[1], ws_ref[:, 128:])
         + b_ref[...])
    nrm = jnp.sqrt(jnp.sum(z * z, axis=1, keepdims=True))
    zn = z / jnp.maximum(nrm, 1e-12)
    act = jnp.maximum(zn, 0.1 * zn)
    out_ref[0] = act[:, :128]
    out_ref[1] = act[:, 128:]


def _conv3_body(a_ref, h_ref, wl_ref, ws_ref, b_ref, b3_ref, wp_ref, bp_ref,
                w0_ref, b0_ref, w1_ref, b1_ref, w2_ref, b2_ref,
                out_ref, sums_ref, cnts_ref, *, bsz, ngraphs, nsteps):
    i = pl.program_id(0)
    z = (_dot_t(a_ref[0], wl_ref[:, :128])
         + _dot_t(a_ref[1], wl_ref[:, 128:])
         + _dot_t(h_ref[0], ws_ref[:, :128])
         + _dot_t(h_ref[1], ws_ref[:, 128:])
         + b_ref[...])
    nrm = jnp.sqrt(jnp.sum(z * z, axis=1, keepdims=True))
    zn = z / jnp.maximum(nrm, 1e-12)
    act = jnp.maximum(zn, 0.1 * zn)
    p = _dot_t(act, wp_ref[...]) + bp_ref[...]
    p = jnp.maximum(p, 0.1 * p)
    b = b3_ref[...].reshape(1, bsz)
    ohT = (lax.broadcasted_iota(jnp.int32, (ngraphs, bsz), 0)
           == jnp.broadcast_to(b, (ngraphs, bsz))).astype(F32)
    s = jnp.dot(ohT, p, preferred_element_type=F32)
    c = jnp.broadcast_to(jnp.sum(ohT, axis=1, keepdims=True), (ngraphs, 128))

    @pl.when(i == 0)
    def _():
        sums_ref[...] = s
        cnts_ref[...] = c

    @pl.when(i > 0)
    def _():
        sums_ref[...] += s
        cnts_ref[...] += c

    @pl.when(i == nsteps - 1)
    def _():
        g = sums_ref[...] / jnp.maximum(cnts_ref[...], 1.0)
        a = jnp.maximum(_dot_t(g, w0_ref[...]) + b0_ref[...], 0.0)
        a = jnp.maximum(_dot_t(a, w1_ref[...]) + b1_ref[...], 0.0)
        zz = _dot_t(a, w2_ref[...]) + b2_ref[...]
        m = jnp.max(zz, axis=1, keepdims=True)
        e = jnp.exp(zz - m)
        out_ref[...] = e / jnp.sum(e, axis=1, keepdims=True)


def _make_segsum(N, E):
    info = plsc.get_sparse_core_info()
    NS = info.num_subcores
    CH = 80
    e_raw = E // NS
    n_ch = -(-e_raw // CH)
    while n_ch % 3 != 2:
        n_ch += 1
    e_pt = n_ch * CH
    NA = N + 16
    rpt = (NA // NS) // 8 * 8
    remz = NA - NS * rpt
    remo = N - NS * rpt
    mesh = plsc.VectorSubcoreMesh(core_axis_name="c", subcore_axis_name="s")
    assert E % NS == 0 and NA % 16 == 0 and remz >= 0 and remo >= 0

    @functools.partial(
        pl.kernel, mesh=mesh,
        out_type=jax.ShapeDtypeStruct((2 * N, 128), F32),
        scratch_types=[
            pltpu.VMEM((e_pt,), jnp.int32),
            pltpu.VMEM((CH,), jnp.int32),
            pltpu.VMEM((CH,), jnp.int32),
            pltpu.VMEM((CH,), jnp.int32),
            pltpu.VMEM((CH, 128), F32),
            pltpu.VMEM((CH, 128), F32),
            pltpu.VMEM((CH, 128), F32),
            pltpu.SemaphoreType.DMA,
            pltpu.SemaphoreType.DMA,
            pltpu.SemaphoreType.DMA,
            pltpu.SemaphoreType.DMA,
            pltpu.SemaphoreType.DMA,
            pltpu.SemaphoreType.DMA,
            pltpu.SemaphoreType.DMA,
            pltpu.SemaphoreType.DMA,
            pltpu.SemaphoreType.DMA,
            pltpu.VMEM_SHARED((NA, 128), F32),
            pltpu.SemaphoreType.DMA,
        ],
    )
    def seg(h_hbm, src_hbm, dst_hbm, zero_hbm, out_hbm,
            srcv, d0, d1, d2, b0, b1, b2,
            sg0, sg1, sg2, ss0, ss1, ss2, sd0, sd1, sd2, acc, sem_z):
        c = lax.axis_index("c")
        s = lax.axis_index("s")
        cN = c * N
        pltpu.async_copy(zero_hbm.at[pl.ds(0, rpt)],
                         acc.at[pl.ds(s * rpt, rpt)], sem_z)

        @pl.when(s == NS - 1)
        def _():
            pltpu.async_copy(zero_hbm.at[pl.ds(0, remz)],
                             acc.at[pl.ds(NS * rpt, remz)], sem_z)

        pltpu.sync_copy(src_hbm.at[pl.ds((c * NS + s) * e_pt, e_pt)], srcv)
        pltpu.make_async_copy(zero_hbm.at[pl.ds(0, rpt)],
                              acc.at[pl.ds(s * rpt, rpt)], sem_z).wait()

        @pl.when(s == NS - 1)
        def _():
            pltpu.make_async_copy(zero_hbm.at[pl.ds(0, remz)],
                                  acc.at[pl.ds(NS * rpt, remz)], sem_z).wait()

        plsc.subcore_barrier()
        dbase = s * e_pt

        def gather(i, buf, sem, dstg, dsem):
            pltpu.async_copy(h_hbm.at[srcv.at[pl.ds(i * CH, CH)]], buf, sem)
            pltpu.async_copy(dst_hbm.at[pl.ds(dbase + i * CH, CH)], dstg, dsem)

        def gwait(buf, sem, dstg, dsem):
            pltpu.make_async_copy(h_hbm.at[pl.ds(0, CH)], buf, sem).wait()
            pltpu.make_async_copy(dst_hbm.at[pl.ds(0, CH)], dstg, dsem).wait()

        def sstart(i, buf, dstg, sem):
            pltpu.async_copy(buf, acc.at[dstg], sem, add=True)

        def swait(buf, sem):
            pltpu.make_async_copy(buf, acc.at[d0], sem).wait()

        K = (n_ch - 2) // 3
        gather(0, b0, sg0, d0, sd0)
        gather(1, b1, sg1, d1, sd1)
        gwait(b0, sg0, d0, sd0)
        sstart(0, b0, d0, ss0)
        gather(2, b2, sg2, d2, sd2)
        gwait(b1, sg1, d1, sd1)
        sstart(1, b1, d1, ss1)

        def body(k, carry):
            j = 3 * k + 2
            swait(b0, ss0)
            gather(j + 1, b0, sg0, d0, sd0)
            gwait(b2, sg2, d2, sd2)
            sstart(j, b2, d2, ss2)
            swait(b1, ss1)
            gather(j + 2, b1, sg1, d1, sd1)
            gwait(b0, sg0, d0, sd0)
            sstart(j + 1, b0, d0, ss0)
            swait(b2, ss2)

            @pl.when(k < K - 1)
            def _():
                gather(j + 3, b2, sg2, d2, sd2)

            gwait(b1, sg1, d1, sd1)
            sstart(j + 2, b1, d1, ss1)
            return carry

        lax.fori_loop(0, K, body, 0)
        swait(b0, ss0)
        swait(b1, ss1)
        plsc.subcore_barrier()
        pltpu.sync_copy(acc.at[pl.ds(s * rpt, rpt)],
                        out_hbm.at[pl.ds(cN + s * rpt, rpt)])

        @pl.when(s == NS - 1)
        def _():
            pltpu.sync_copy(acc.at[pl.ds(NS * rpt, remo)],
                            out_hbm.at[pl.ds(cN + NS * rpt, remo)])

    def prep(src, dst):
        pad = e_pt - e_raw
        spread = (jnp.arange(NS, dtype=jnp.int32)[:, None] * max(pad, 1)
                  + jnp.arange(pad, dtype=jnp.int32)[None, :]) % N
        src_p = jnp.concatenate([src.reshape(NS, e_raw), spread],
                                axis=1).reshape(-1)
        src_p = jnp.concatenate([src_p, src_p + N])
        dummy = (N + jnp.arange(NS, dtype=jnp.int32))[:, None]
        dst_p = jnp.concatenate(
            [dst.reshape(NS, e_raw),
             jnp.broadcast_to(dummy, (NS, pad))], axis=1).reshape(-1)
        return src_p, dst_p

    return seg, prep


def kernel(x, edge_index, batch, W_emb, b_emb, Wl0, Wself0, bconv0,
           Wl1, Wself1, bconv1, Wl2, Wself2, bconv2, W_p, b_p,
           W_a0, b_a0, W_a1, b_a1, W_a2, b_a2):
    N, D = x.shape
    E = edge_index.shape[1]
    G = 32
    NC = W_a2.shape[0]
    B = 1000
    grid = N // B

    src = edge_index[0]
    dst = edge_index[1]
    zeros_blk = jnp.zeros(((N // 16) // 8 * 8, 128), F32)

    h_split = pl.pallas_call(
        _emb_body,
        grid=(grid,),
        in_specs=[
            pl.BlockSpec((B, D), lambda i: (i, 0)),
            pl.BlockSpec((D, D), lambda i: (0, 0)),
            pl.BlockSpec((1, D), lambda i: (0, 0)),
        ],
        out_specs=pl.BlockSpec((2, B, 128), lambda i: (0, i, 0)),
        out_shape=jax.ShapeDtypeStruct((2, N, 128), F32),
    )(x, W_emb, b_emb.reshape(1, D))

    segsum, seg_prep = _make_segsum(N, E)
    src_p, dst_p = seg_prep(src, dst)

    conv_call = pl.pallas_call(
        _conv_body,
        grid=(grid,),
        in_specs=[
            pl.BlockSpec((2, B, 128), lambda i: (0, i, 0)),
            pl.BlockSpec((2, B, 128), lambda i: (0, i, 0)),
            pl.BlockSpec((D, D), lambda i: (0, 0)),
            pl.BlockSpec((D, D), lambda i: (0, 0)),
            pl.BlockSpec((1, D), lambda i: (0, 0)),
        ],
        out_specs=pl.BlockSpec((2, B, 128), lambda i: (0, i, 0)),
        out_shape=jax.ShapeDtypeStruct((2, N, 128), F32),
    )

    for Wl, Ws, bc in ((Wl0, Wself0, bconv0), (Wl1, Wself1, bconv1)):
        agg2 = segsum(h_split.reshape(2 * N, 128), src_p, dst_p, zeros_blk)
        h_split = conv_call(
            agg2.reshape(2, N, 128), h_split, Wl, Ws, bc.reshape(1, D))

    agg2 = segsum(h_split.reshape(2 * N, 128), src_p, dst_p, zeros_blk)
    A0, A1 = W_a0.shape[0], W_a1.shape[0]
    out = pl.pallas_call(
        functools.partial(_conv3_body, bsz=B, ngraphs=G, nsteps=grid),
        grid=(grid,),
        in_specs=[
            pl.BlockSpec((2, B, 128), lambda i: (0, i, 0)),
            pl.BlockSpec((2, B, 128), lambda i: (0, i, 0)),
            pl.BlockSpec((D, D), lambda i: (0, 0)),
            pl.BlockSpec((D, D), lambda i: (0, 0)),
            pl.BlockSpec((1, D), lambda i: (0, 0)),
            pl.BlockSpec((1, 1, B), lambda i: (i, 0, 0)),
            pl.BlockSpec((128, D), lambda i: (0, 0)),
            pl.BlockSpec((1, 128), lambda i: (0, 0)),
            pl.BlockSpec((A0, 128), lambda i: (0, 0)),
            pl.BlockSpec((1, A0), lambda i: (0, 0)),
            pl.BlockSpec((A1, A0), lambda i: (0, 0)),
            pl.BlockSpec((1, A1), lambda i: (0, 0)),
            pl.BlockSpec((NC, A1), lambda i: (0, 0)),
            pl.BlockSpec((1, NC), lambda i: (0, 0)),
        ],
        out_specs=pl.BlockSpec((G, NC), lambda i: (0, 0)),
        out_shape=jax.ShapeDtypeStruct((G, NC), F32),
        scratch_shapes=[
            pltpu.VMEM((G, 128), F32),
            pltpu.VMEM((G, 128), F32),
        ],
    )(agg2.reshape(2, N, 128), h_split, Wl2, Wself2, bconv2.reshape(1, D),
      batch.reshape(grid, 1, B), W_p, b_p.reshape(1, 128),
      W_a0, b_a0.reshape(1, A0), W_a1, b_a1.reshape(1, A1),
      W_a2, b_a2.reshape(1, NC))

    return out


# speedup vs baseline: 1.4888x; 1.0218x over previous
import functools

import jax
import jax.numpy as jnp
from jax import lax
from jax.experimental import pallas as pl
from jax.experimental.pallas import tpu as pltpu
from jax.experimental.pallas import tpu_sc as plsc

F32 = jnp.float32


def _dot_t(a, w):
    return lax.dot_general(a, w, (((1,), (1,)), ((), ())),
                           preferred_element_type=F32)


def _emb_body(x_ref, w_ref, b_ref, out_ref):
    z = _dot_t(x_ref[...], w_ref[...]) + b_ref[...]
    out_ref[0] = z[:, :128]
    out_ref[1] = z[:, 128:]


def _conv_body(a_ref, h_ref, wl_ref, ws_ref, b_ref, out_ref):
    z = (_dot_t(a_ref[0], wl_ref[:, :128])
         + _dot_t(a_ref[1], wl_ref[:, 128:])
         + _dot_t(h_ref[0], ws_ref[:, :128])
         + _dot_t(h_ref[1], ws_ref[:, 128:])
         + b_ref[...])
    nrm = jnp.sqrt(jnp.sum(z * z, axis=1, keepdims=True))
    zn = z / jnp.maximum(nrm, 1e-12)
    act = jnp.maximum(zn, 0.1 * zn)
    out_ref[0] = act[:, :128]
    out_ref[1] = act[:, 128:]


def _conv3_body(a_ref, h_ref, wl_ref, ws_ref, b_ref, b3_ref, wp_ref, bp_ref,
                w0_ref, b0_ref, w1_ref, b1_ref, w2_ref, b2_ref,
                out_ref, sums_ref, cnts_ref, *, bsz, ngraphs, nsteps):
    i = pl.program_id(0)
    z = (_dot_t(a_ref[0], wl_ref[:, :128])
         + _dot_t(a_ref[1], wl_ref[:, 128:])
         + _dot_t(h_ref[0], ws_ref[:, :128])
         + _dot_t(h_ref[1], ws_ref[:, 128:])
         + b_ref[...])
    nrm = jnp.sqrt(jnp.sum(z * z, axis=1, keepdims=True))
    zn = z / jnp.maximum(nrm, 1e-12)
    act = jnp.maximum(zn, 0.1 * zn)
    p = _dot_t(act, wp_ref[...]) + bp_ref[...]
    p = jnp.maximum(p, 0.1 * p)
    b = b3_ref[...].reshape(1, bsz)
    ohT = (lax.broadcasted_iota(jnp.int32, (ngraphs, bsz), 0)
           == jnp.broadcast_to(b, (ngraphs, bsz))).astype(F32)
    s = jnp.dot(ohT, p, preferred_element_type=F32)
    c = jnp.broadcast_to(jnp.sum(ohT, axis=1, keepdims=True), (ngraphs, 128))

    @pl.when(i == 0)
    def _():
        sums_ref[...] = s
        cnts_ref[...] = c

    @pl.when(i > 0)
    def _():
        sums_ref[...] += s
        cnts_ref[...] += c

    @pl.when(i == nsteps - 1)
    def _():
        g = sums_ref[...] / jnp.maximum(cnts_ref[...], 1.0)
        a = jnp.maximum(_dot_t(g, w0_ref[...]) + b0_ref[...], 0.0)
        a = jnp.maximum(_dot_t(a, w1_ref[...]) + b1_ref[...], 0.0)
        zz = _dot_t(a, w2_ref[...]) + b2_ref[...]
        m = jnp.max(zz, axis=1, keepdims=True)
        e = jnp.exp(zz - m)
        out_ref[...] = e / jnp.sum(e, axis=1, keepdims=True)


def _make_segsum(N, E):
    info = plsc.get_sparse_core_info()
    NS = info.num_subcores
    CH = 80
    e_raw = E // NS
    n_ch = -(-e_raw // CH)
    while n_ch % 3 != 2:
        n_ch += 1
    e_pt = n_ch * CH
    NA = N + 16
    rpt = (NA // NS) // 8 * 8
    remz = NA - NS * rpt
    remo = N - NS * rpt
    mesh = plsc.VectorSubcoreMesh(core_axis_name="c", subcore_axis_name="s")
    assert E % NS == 0 and NA % 16 == 0 and remz >= 0 and remo >= 0

    @functools.partial(
        pl.kernel, mesh=mesh,
        out_type=jax.ShapeDtypeStruct((2 * N, 128), F32),
        scratch_types=[
            pltpu.VMEM((e_pt,), jnp.int32),
            pltpu.VMEM((CH,), jnp.int32),
            pltpu.VMEM((CH,), jnp.int32),
            pltpu.VMEM((CH,), jnp.int32),
            pltpu.VMEM((CH, 128), F32),
            pltpu.VMEM((CH, 128), F32),
            pltpu.VMEM((CH, 128), F32),
            pltpu.SemaphoreType.DMA,
            pltpu.SemaphoreType.DMA,
            pltpu.SemaphoreType.DMA,
            pltpu.SemaphoreType.DMA,
            pltpu.SemaphoreType.DMA,
            pltpu.SemaphoreType.DMA,
            pltpu.SemaphoreType.DMA,
            pltpu.SemaphoreType.DMA,
            pltpu.SemaphoreType.DMA,
            pltpu.VMEM_SHARED((NA, 128), F32),
            pltpu.SemaphoreType.DMA,
        ],
    )
    def seg(h_hbm, src_hbm, dst_hbm, zero_hbm, out_hbm,
            srcv, d0, d1, d2, b0, b1, b2,
            sg0, sg1, sg2, ss0, ss1, ss2, sd0, sd1, sd2, acc, sem_z):
        c = lax.axis_index("c")
        s = lax.axis_index("s")
        cN = c * N
        pltpu.async_copy(zero_hbm.at[pl.ds(0, rpt)],
                         acc.at[pl.ds(s * rpt, rpt)], sem_z)

        @pl.when(s == NS - 1)
        def _():
            pltpu.async_copy(zero_hbm.at[pl.ds(0, remz)],
                             acc.at[pl.ds(NS * rpt, remz)], sem_z)

        pltpu.sync_copy(src_hbm.at[pl.ds((c * NS + s) * e_pt, e_pt)], srcv)
        pltpu.make_async_copy(zero_hbm.at[pl.ds(0, rpt)],
                              acc.at[pl.ds(s * rpt, rpt)], sem_z).wait()

        @pl.when(s == NS - 1)
        def _():
            pltpu.make_async_copy(zero_hbm.at[pl.ds(0, remz)],
                                  acc.at[pl.ds(NS * rpt, remz)], sem_z).wait()

        plsc.subcore_barrier()
        dbase = s * e_pt

        def gather(i, buf, sem, dstg, dsem):
            pltpu.async_copy(h_hbm.at[srcv.at[pl.ds(i * CH, CH)]], buf, sem)
            pltpu.async_copy(dst_hbm.at[pl.ds(dbase + i * CH, CH)], dstg, dsem)

        def gwait(buf, sem, dstg, dsem):
            pltpu.make_async_copy(h_hbm.at[pl.ds(0, CH)], buf, sem).wait()
            pltpu.make_async_copy(dst_hbm.at[pl.ds(0, CH)], dstg, dsem).wait()

        def sstart(i, buf, dstg, sem):
            pltpu.async_copy(buf, acc.at[dstg], sem, add=True)

        def swait(buf, sem):
            pltpu.make_async_copy(buf, acc.at[d0], sem).wait()

        K = (n_ch - 2) // 3
        gather(0, b0, sg0, d0, sd0)
        gather(1, b1, sg1, d1, sd1)
        gwait(b0, sg0, d0, sd0)
        sstart(0, b0, d0, ss0)
        gather(2, b2, sg2, d2, sd2)
        gwait(b1, sg1, d1, sd1)
        sstart(1, b1, d1, ss1)

        def body(k, carry):
            j = 3 * k + 2
            swait(b0, ss0)
            gather(j + 1, b0, sg0, d0, sd0)
            gwait(b2, sg2, d2, sd2)
            sstart(j, b2, d2, ss2)
            swait(b1, ss1)
            gather(j + 2, b1, sg1, d1, sd1)
            gwait(b0, sg0, d0, sd0)
            sstart(j + 1, b0, d0, ss0)
            swait(b2, ss2)

            @pl.when(k < K - 1)
            def _():
                gather(j + 3, b2, sg2, d2, sd2)

            gwait(b1, sg1, d1, sd1)
            sstart(j + 2, b1, d1, ss1)
            return carry

        lax.fori_loop(0, K, body, 0)
        swait(b0, ss0)
        swait(b1, ss1)
        plsc.subcore_barrier()
        pltpu.sync_copy(acc.at[pl.ds(s * rpt, rpt)],
                        out_hbm.at[pl.ds(cN + s * rpt, rpt)])

        @pl.when(s == NS - 1)
        def _():
            pltpu.sync_copy(acc.at[pl.ds(NS * rpt, remo)],
                            out_hbm.at[pl.ds(cN + NS * rpt, remo)])

    def prep(src, dst):
        pad = e_pt - e_raw
        spread = (jnp.arange(NS, dtype=jnp.int32)[:, None] * max(pad, 1)
                  + jnp.arange(pad, dtype=jnp.int32)[None, :]) % N
        src_p = jnp.concatenate([src.reshape(NS, e_raw), spread],
                                axis=1).reshape(-1)
        src_p = jnp.concatenate([src_p, src_p + N])
        dummy = (N + jnp.arange(NS, dtype=jnp.int32))[:, None]
        dst_p = jnp.concatenate(
            [dst.reshape(NS, e_raw),
             jnp.broadcast_to(dummy, (NS, pad))], axis=1).reshape(-1)
        return src_p, dst_p

    return seg, prep


def kernel(x, edge_index, batch, W_emb, b_emb, Wl0, Wself0, bconv0,
           Wl1, Wself1, bconv1, Wl2, Wself2, bconv2, W_p, b_p,
           W_a0, b_a0, W_a1, b_a1, W_a2, b_a2):
    N, D = x.shape
    E = edge_index.shape[1]
    G = 32
    NC = W_a2.shape[0]
    B = 2000
    grid = N // B

    src = edge_index[0]
    dst = edge_index[1]
    zeros_blk = jnp.zeros(((N // 16) // 8 * 8, 128), F32)

    h_split = pl.pallas_call(
        _emb_body,
        grid=(grid,),
        in_specs=[
            pl.BlockSpec((B, D), lambda i: (i, 0)),
            pl.BlockSpec((D, D), lambda i: (0, 0)),
            pl.BlockSpec((1, D), lambda i: (0, 0)),
        ],
        out_specs=pl.BlockSpec((2, B, 128), lambda i: (0, i, 0)),
        out_shape=jax.ShapeDtypeStruct((2, N, 128), F32),
    )(x, W_emb, b_emb.reshape(1, D))

    segsum, seg_prep = _make_segsum(N, E)
    src_p, dst_p = seg_prep(src, dst)

    conv_call = pl.pallas_call(
        _conv_body,
        grid=(grid,),
        in_specs=[
            pl.BlockSpec((2, B, 128), lambda i: (0, i, 0)),
            pl.BlockSpec((2, B, 128), lambda i: (0, i, 0)),
            pl.BlockSpec((D, D), lambda i: (0, 0)),
            pl.BlockSpec((D, D), lambda i: (0, 0)),
            pl.BlockSpec((1, D), lambda i: (0, 0)),
        ],
        out_specs=pl.BlockSpec((2, B, 128), lambda i: (0, i, 0)),
        out_shape=jax.ShapeDtypeStruct((2, N, 128), F32),
    )

    for Wl, Ws, bc in ((Wl0, Wself0, bconv0), (Wl1, Wself1, bconv1)):
        agg2 = segsum(h_split.reshape(2 * N, 128), src_p, dst_p, zeros_blk)
        h_split = conv_call(
            agg2.reshape(2, N, 128), h_split, Wl, Ws, bc.reshape(1, D))

    agg2 = segsum(h_split.reshape(2 * N, 128), src_p, dst_p, zeros_blk)
    A0, A1 = W_a0.shape[0], W_a1.shape[0]
    out = pl.pallas_call(
        functools.partial(_conv3_body, bsz=B, ngraphs=G, nsteps=grid),
        grid=(grid,),
        in_specs=[
            pl.BlockSpec((2, B, 128), lambda i: (0, i, 0)),
            pl.BlockSpec((2, B, 128), lambda i: (0, i, 0)),
            pl.BlockSpec((D, D), lambda i: (0, 0)),
            pl.BlockSpec((D, D), lambda i: (0, 0)),
            pl.BlockSpec((1, D), lambda i: (0, 0)),
            pl.BlockSpec((1, 1, B), lambda i: (i, 0, 0)),
            pl.BlockSpec((128, D), lambda i: (0, 0)),
            pl.BlockSpec((1, 128), lambda i: (0, 0)),
            pl.BlockSpec((A0, 128), lambda i: (0, 0)),
            pl.BlockSpec((1, A0), lambda i: (0, 0)),
            pl.BlockSpec((A1, A0), lambda i: (0, 0)),
            pl.BlockSpec((1, A1), lambda i: (0, 0)),
            pl.BlockSpec((NC, A1), lambda i: (0, 0)),
            pl.BlockSpec((1, NC), lambda i: (0, 0)),
        ],
        out_specs=pl.BlockSpec((G, NC), lambda i: (0, 0)),
        out_shape=jax.ShapeDtypeStruct((G, NC), F32),
        scratch_shapes=[
            pltpu.VMEM((G, 128), F32),
            pltpu.VMEM((G, 128), F32),
        ],
    )(agg2.reshape(2, N, 128), h_split, Wl2, Wself2, bconv2.reshape(1, D),
      batch.reshape(grid, 1, B), W_p, b_p.reshape(1, 128),
      W_a0, b_a0.reshape(1, A0), W_a1, b_a1.reshape(1, A1),
      W_a2, b_a2.reshape(1, NC))

    return out


# speedup vs baseline: 1.5047x; 1.0107x over previous
import functools

import jax
import jax.numpy as jnp
from jax import lax
from jax.experimental import pallas as pl
from jax.experimental.pallas import tpu as pltpu
from jax.experimental.pallas import tpu_sc as plsc

F32 = jnp.float32


def _dot_t(a, w):
    return lax.dot_general(a, w, (((1,), (1,)), ((), ())),
                           preferred_element_type=F32)


def _emb_body(x_ref, w_ref, b_ref, out_ref):
    z = _dot_t(x_ref[...], w_ref[...]) + b_ref[...]
    out_ref[0] = z[:, :128]
    out_ref[1] = z[:, 128:]


def _conv_body(a_ref, h_ref, wl_ref, ws_ref, b_ref, out_ref):
    z = (_dot_t(a_ref[0], wl_ref[:, :128])
         + _dot_t(a_ref[1], wl_ref[:, 128:])
         + _dot_t(h_ref[0], ws_ref[:, :128])
         + _dot_t(h_ref[1], ws_ref[:, 128:])
         + b_ref[...])
    nrm = jnp.sqrt(jnp.sum(z * z, axis=1, keepdims=True))
    zn = z / jnp.maximum(nrm, 1e-12)
    act = jnp.maximum(zn, 0.1 * zn)
    out_ref[0] = act[:, :128]
    out_ref[1] = act[:, 128:]


def _conv3_body(a_ref, h_ref, wl_ref, ws_ref, b_ref, b3_ref, wp_ref, bp_ref,
                w0_ref, b0_ref, w1_ref, b1_ref, w2_ref, b2_ref,
                out_ref, sums_ref, cnts_ref, *, bsz, ngraphs, nsteps):
    i = pl.program_id(0)
    z = (_dot_t(a_ref[0], wl_ref[:, :128])
         + _dot_t(a_ref[1], wl_ref[:, 128:])
         + _dot_t(h_ref[0], ws_ref[:, :128])
         + _dot_t(h_ref[1], ws_ref[:, 128:])
         + b_ref[...])
    nrm = jnp.sqrt(jnp.sum(z * z, axis=1, keepdims=True))
    zn = z / jnp.maximum(nrm, 1e-12)
    act = jnp.maximum(zn, 0.1 * zn)
    p = _dot_t(act, wp_ref[...]) + bp_ref[...]
    p = jnp.maximum(p, 0.1 * p)
    b = b3_ref[...].reshape(1, bsz)
    ohT = (lax.broadcasted_iota(jnp.int32, (ngraphs, bsz), 0)
           == jnp.broadcast_to(b, (ngraphs, bsz))).astype(F32)
    s = jnp.dot(ohT, p, preferred_element_type=F32)
    c = jnp.broadcast_to(jnp.sum(ohT, axis=1, keepdims=True), (ngraphs, 128))

    @pl.when(i == 0)
    def _():
        sums_ref[...] = s
        cnts_ref[...] = c

    @pl.when(i > 0)
    def _():
        sums_ref[...] += s
        cnts_ref[...] += c

    @pl.when(i == nsteps - 1)
    def _():
        g = sums_ref[...] / jnp.maximum(cnts_ref[...], 1.0)
        a = jnp.maximum(_dot_t(g, w0_ref[...]) + b0_ref[...], 0.0)
        a = jnp.maximum(_dot_t(a, w1_ref[...]) + b1_ref[...], 0.0)
        zz = _dot_t(a, w2_ref[...]) + b2_ref[...]
        m = jnp.max(zz, axis=1, keepdims=True)
        e = jnp.exp(zz - m)
        out_ref[...] = e / jnp.sum(e, axis=1, keepdims=True)


def _make_segsum(N, E):
    info = plsc.get_sparse_core_info()
    NS = info.num_subcores
    CH = 80
    e_raw = E // NS
    n_ch = -(-e_raw // CH)
    while n_ch % 3 != 2:
        n_ch += 1
    e_pt = n_ch * CH
    NA = N + 16
    rpt = (NA // NS) // 8 * 8
    remz = NA - NS * rpt
    remo = N - NS * rpt
    mesh = plsc.VectorSubcoreMesh(core_axis_name="c", subcore_axis_name="s")
    assert E % NS == 0 and NA % 16 == 0 and remz >= 0 and remo >= 0

    @functools.partial(
        pl.kernel, mesh=mesh,
        out_type=jax.ShapeDtypeStruct((2 * N, 128), F32),
        scratch_types=[
            pltpu.VMEM((e_pt,), jnp.int32),
            pltpu.VMEM((CH,), jnp.int32),
            pltpu.VMEM((CH,), jnp.int32),
            pltpu.VMEM((CH,), jnp.int32),
            pltpu.VMEM((CH, 128), F32),
            pltpu.VMEM((CH, 128), F32),
            pltpu.VMEM((CH, 128), F32),
            pltpu.SemaphoreType.DMA,
            pltpu.SemaphoreType.DMA,
            pltpu.SemaphoreType.DMA,
            pltpu.SemaphoreType.DMA,
            pltpu.SemaphoreType.DMA,
            pltpu.SemaphoreType.DMA,
            pltpu.SemaphoreType.DMA,
            pltpu.SemaphoreType.DMA,
            pltpu.SemaphoreType.DMA,
            pltpu.VMEM_SHARED((NA, 128), F32),
            pltpu.SemaphoreType.DMA,
        ],
    )
    def seg(h_hbm, src_hbm, dst_hbm, zero_hbm, out_hbm,
            srcv, d0, d1, d2, b0, b1, b2,
            sg0, sg1, sg2, ss0, ss1, ss2, sd0, sd1, sd2, acc, sem_z):
        c = lax.axis_index("c")
        s = lax.axis_index("s")
        cN = c * N
        pltpu.async_copy(zero_hbm.at[pl.ds(0, rpt)],
                         acc.at[pl.ds(s * rpt, rpt)], sem_z)

        @pl.when(s == NS - 1)
        def _():
            pltpu.async_copy(zero_hbm.at[pl.ds(0, remz)],
                             acc.at[pl.ds(NS * rpt, remz)], sem_z)

        pltpu.sync_copy(src_hbm.at[pl.ds((c * NS + s) * e_pt, e_pt)], srcv)
        pltpu.make_async_copy(zero_hbm.at[pl.ds(0, rpt)],
                              acc.at[pl.ds(s * rpt, rpt)], sem_z).wait()

        @pl.when(s == NS - 1)
        def _():
            pltpu.make_async_copy(zero_hbm.at[pl.ds(0, remz)],
                                  acc.at[pl.ds(NS * rpt, remz)], sem_z).wait()

        plsc.subcore_barrier()
        dbase = s * e_pt

        def gather(i, buf, sem, dstg, dsem):
            pltpu.async_copy(h_hbm.at[srcv.at[pl.ds(i * CH, CH)]], buf, sem)
            pltpu.async_copy(dst_hbm.at[pl.ds(dbase + i * CH, CH)], dstg, dsem)

        def gwait(buf, sem, dstg, dsem):
            pltpu.make_async_copy(h_hbm.at[pl.ds(0, CH)], buf, sem).wait()
            pltpu.make_async_copy(dst_hbm.at[pl.ds(0, CH)], dstg, dsem).wait()

        def sstart(i, buf, dstg, sem):
            pltpu.async_copy(buf, acc.at[dstg], sem, add=True)

        def swait(buf, sem):
            pltpu.make_async_copy(buf, acc.at[d0], sem).wait()

        K = (n_ch - 2) // 3
        gather(0, b0, sg0, d0, sd0)
        gather(1, b1, sg1, d1, sd1)
        gwait(b0, sg0, d0, sd0)
        sstart(0, b0, d0, ss0)
        gather(2, b2, sg2, d2, sd2)
        gwait(b1, sg1, d1, sd1)
        sstart(1, b1, d1, ss1)

        def body(k, carry):
            j = 3 * k + 2
            swait(b0, ss0)
            gather(j + 1, b0, sg0, d0, sd0)
            gwait(b2, sg2, d2, sd2)
            sstart(j, b2, d2, ss2)
            swait(b1, ss1)
            gather(j + 2, b1, sg1, d1, sd1)
            gwait(b0, sg0, d0, sd0)
            sstart(j + 1, b0, d0, ss0)
            swait(b2, ss2)

            @pl.when(k < K - 1)
            def _():
                gather(j + 3, b2, sg2, d2, sd2)

            gwait(b1, sg1, d1, sd1)
            sstart(j + 2, b1, d1, ss1)
            return carry

        lax.fori_loop(0, K, body, 0)
        swait(b0, ss0)
        swait(b1, ss1)
        plsc.subcore_barrier()
        pltpu.sync_copy(acc.at[pl.ds(s * rpt, rpt)],
                        out_hbm.at[pl.ds(cN + s * rpt, rpt)])

        @pl.when(s == NS - 1)
        def _():
            pltpu.sync_copy(acc.at[pl.ds(NS * rpt, remo)],
                            out_hbm.at[pl.ds(cN + NS * rpt, remo)])

    def prep(src, dst):
        pad = e_pt - e_raw
        spread = (jnp.arange(NS, dtype=jnp.int32)[:, None] * max(pad, 1)
                  + jnp.arange(pad, dtype=jnp.int32)[None, :]) % N
        src_p = jnp.concatenate([src.reshape(NS, e_raw), spread],
                                axis=1).reshape(-1)
        src_p = jnp.concatenate([src_p, src_p + N])
        dummy = (N + jnp.arange(NS, dtype=jnp.int32))[:, None]
        dst_p = jnp.concatenate(
            [dst.reshape(NS, e_raw),
             jnp.broadcast_to(dummy, (NS, pad))], axis=1).reshape(-1)
        return src_p, dst_p

    return seg, prep


def kernel(x, edge_index, batch, W_emb, b_emb, Wl0, Wself0, bconv0,
           Wl1, Wself1, bconv1, Wl2, Wself2, bconv2, W_p, b_p,
           W_a0, b_a0, W_a1, b_a1, W_a2, b_a2):
    N, D = x.shape
    E = edge_index.shape[1]
    G = 32
    NC = W_a2.shape[0]
    B = 5000
    grid = N // B

    src = edge_index[0]
    dst = edge_index[1]
    zeros_blk = jnp.zeros(((N // 16) // 8 * 8, 128), F32)

    h_split = pl.pallas_call(
        _emb_body,
        grid=(grid,),
        in_specs=[
            pl.BlockSpec((B, D), lambda i: (i, 0)),
            pl.BlockSpec((D, D), lambda i: (0, 0)),
            pl.BlockSpec((1, D), lambda i: (0, 0)),
        ],
        out_specs=pl.BlockSpec((2, B, 128), lambda i: (0, i, 0)),
        out_shape=jax.ShapeDtypeStruct((2, N, 128), F32),
    )(x, W_emb, b_emb.reshape(1, D))

    segsum, seg_prep = _make_segsum(N, E)
    src_p, dst_p = seg_prep(src, dst)

    conv_call = pl.pallas_call(
        _conv_body,
        grid=(grid,),
        in_specs=[
            pl.BlockSpec((2, B, 128), lambda i: (0, i, 0)),
            pl.BlockSpec((2, B, 128), lambda i: (0, i, 0)),
            pl.BlockSpec((D, D), lambda i: (0, 0)),
            pl.BlockSpec((D, D), lambda i: (0, 0)),
            pl.BlockSpec((1, D), lambda i: (0, 0)),
        ],
        out_specs=pl.BlockSpec((2, B, 128), lambda i: (0, i, 0)),
        out_shape=jax.ShapeDtypeStruct((2, N, 128), F32),
    )

    for Wl, Ws, bc in ((Wl0, Wself0, bconv0), (Wl1, Wself1, bconv1)):
        agg2 = segsum(h_split.reshape(2 * N, 128), src_p, dst_p, zeros_blk)
        h_split = conv_call(
            agg2.reshape(2, N, 128), h_split, Wl, Ws, bc.reshape(1, D))

    agg2 = segsum(h_split.reshape(2 * N, 128), src_p, dst_p, zeros_blk)
    A0, A1 = W_a0.shape[0], W_a1.shape[0]
    out = pl.pallas_call(
        functools.partial(_conv3_body, bsz=B, ngraphs=G, nsteps=grid),
        grid=(grid,),
        in_specs=[
            pl.BlockSpec((2, B, 128), lambda i: (0, i, 0)),
            pl.BlockSpec((2, B, 128), lambda i: (0, i, 0)),
            pl.BlockSpec((D, D), lambda i: (0, 0)),
            pl.BlockSpec((D, D), lambda i: (0, 0)),
            pl.BlockSpec((1, D), lambda i: (0, 0)),
            pl.BlockSpec((1, 1, B), lambda i: (i, 0, 0)),
            pl.BlockSpec((128, D), lambda i: (0, 0)),
            pl.BlockSpec((1, 128), lambda i: (0, 0)),
            pl.BlockSpec((A0, 128), lambda i: (0, 0)),
            pl.BlockSpec((1, A0), lambda i: (0, 0)),
            pl.BlockSpec((A1, A0), lambda i: (0, 0)),
            pl.BlockSpec((1, A1), lambda i: (0, 0)),
            pl.BlockSpec((NC, A1), lambda i: (0, 0)),
            pl.BlockSpec((1, NC), lambda i: (0, 0)),
        ],
        out_specs=pl.BlockSpec((G, NC), lambda i: (0, 0)),
        out_shape=jax.ShapeDtypeStruct((G, NC), F32),
        scratch_shapes=[
            pltpu.VMEM((G, 128), F32),
            pltpu.VMEM((G, 128), F32),
        ],
    )(agg2.reshape(2, N, 128), h_split, Wl2, Wself2, bconv2.reshape(1, D),
      batch.reshape(grid, 1, B), W_p, b_p.reshape(1, 128),
      W_a0, b_a0.reshape(1, A0), W_a1, b_a1.reshape(1, A1),
      W_a2, b_a2.reshape(1, NC))

    return out
